# baseline (speedup 1.0000x reference)
import jax
import jax.numpy as jnp
from jax import lax
from jax.experimental import pallas as pl
from jax.experimental.pallas import tpu as pltpu
from jax.experimental.pallas import tpu_sc as plsc

N, H, E, K, I = 2048, 1024, 16, 2, 512
NSLOT = N * K
P0 = NSLOT + 256
NTOT = P0 + N
BLK = 256
PADN = NTOT + BLK
HW = H // 2
NEG = -1e30
INTMAX = 2147483647


def _gate_body(x_ref, gw_ref, pos_ref, w_ref, offs_ref):
    logits = lax.dot_general(x_ref[...], gw_ref[...], (((1,), (1,)), ((), ())),
                             preferred_element_type=jnp.float32)
    iota = lax.broadcasted_iota(jnp.int32, (N, 128), 1)
    l = jnp.where(iota < E, logits, NEG)
    m = jnp.max(l, axis=1, keepdims=True)
    z = jnp.sum(jnp.exp(l - m), axis=1, keepdims=True)
    a1 = jnp.min(jnp.where(l == m, iota, INTMAX), axis=1, keepdims=True)
    l2 = jnp.where(iota == a1, NEG, l)
    m2 = jnp.max(l2, axis=1, keepdims=True)
    a2 = jnp.min(jnp.where(l2 == m2, iota, INTMAX), axis=1, keepdims=True)
    s1 = 1.0 / z
    s2 = jnp.exp(m2 - m) / z
    denom = s1 + s2 + 1e-20
    w_ref[...] = jnp.concatenate([s1 / denom, s2 / denom], axis=1)

    ecol = lax.broadcasted_iota(jnp.int32, (N, E), 1)
    oh1 = (a1 == ecol).astype(jnp.float32)
    oh2 = (a2 == ecol).astype(jnp.float32)
    tri_r = lax.broadcasted_iota(jnp.int32, (N, N), 0)
    tri_c = lax.broadcasted_iota(jnp.int32, (N, N), 1)
    stl = (tri_c < tri_r).astype(jnp.float32)
    pre1 = lax.dot_general(stl, oh1, (((1,), (0,)), ((), ())),
                           preferred_element_type=jnp.float32)
    pre2 = lax.dot_general(stl, oh2, (((1,), (0,)), ((), ())),
                           preferred_element_type=jnp.float32)
    tot1 = jnp.sum(oh1, axis=0, keepdims=True)
    tot2 = jnp.sum(oh2, axis=0, keepdims=True)
    counts = tot1 + tot2
    c16 = jnp.float32(16.0) * jnp.ceil(counts / 16.0)
    e_r = lax.broadcasted_iota(jnp.int32, (E, E), 0)
    e_c = lax.broadcasted_iota(jnp.int32, (E, E), 1)
    stl16 = (e_r < e_c).astype(jnp.float32)
    excl = lax.dot_general(c16, stl16, (((1,), (0,)), ((), ())),
                           preferred_element_type=jnp.float32)
    rank1 = jnp.sum(oh1 * pre1, axis=1, keepdims=True)
    rank2 = jnp.sum(oh2 * pre2, axis=1, keepdims=True)
    base1 = jnp.sum(oh1 * excl, axis=1, keepdims=True)
    base2 = jnp.sum(oh2 * (excl + tot1), axis=1, keepdims=True)
    pos1 = (base1 + rank1).astype(jnp.int32)
    pos2 = (base2 + rank2).astype(jnp.int32)
    pos_ref[...] = jnp.concatenate([pos1, pos2], axis=1)
    offs_ref[...] = jnp.broadcast_to(excl, (8, E)).astype(jnp.int32)


def _gate(x, gwp):
    return pl.pallas_call(
        _gate_body,
        in_specs=[pl.BlockSpec((N, H), lambda: (0, 0)),
                  pl.BlockSpec((128, H), lambda: (0, 0))],
        out_specs=[pl.BlockSpec((N, 2), lambda: (0, 0)),
                   pl.BlockSpec((N, 2), lambda: (0, 0)),
                   pl.BlockSpec((8, E), lambda: (0, 0))],
        out_shape=[jax.ShapeDtypeStruct((N, 2), jnp.int32),
                   jax.ShapeDtypeStruct((N, 2), jnp.float32),
                   jax.ShapeDtypeStruct((8, E), jnp.int32)],
    )(x, gwp)


_MESH_CACHE = []


def _sc_mesh():
    if not _MESH_CACHE:
        _MESH_CACHE.append(
            plsc.VectorSubcoreMesh(core_axis_name="c", subcore_axis_name="s"))
    return _MESH_CACHE[0]


def _dist_body(pos, ws, xbits, xs, wsrt, pos_loc, ws_loc, xloc, ones_loc, sem):
    cid = lax.axis_index("c")
    sid = lax.axis_index("s")
    w = cid * 16 + sid
    sbase = w * 128
    tok0 = lax.rem(w, 16) * 128

    pltpu.sync_copy(pos.at[pl.ds(sbase, 128)], pos_loc)
    pltpu.sync_copy(ws.at[pl.ds(sbase, 128)], ws_loc)
    pltpu.sync_copy(xbits.at[pl.ds(tok0, 128)], xloc)

    pltpu.async_copy(xloc, xs.at[pos_loc], sem).wait()
    pltpu.async_copy(ws_loc, wsrt.at[pos_loc], sem).wait()

    @pl.when(w >= 16)
    def _():
        for j in range(8):
            ones_loc[pl.ds(j * 16, 16)] = jnp.full((16,), 1.0, jnp.float32)
        pltpu.sync_copy(xloc, xs.at[pl.ds(P0 + tok0, 128)])
        pltpu.sync_copy(ones_loc, wsrt.at[pl.ds(P0 + tok0, 128)])


def _distribute(pos_cm, ws_cm, xbits):
    f = pl.kernel(
        _dist_body,
        mesh=_sc_mesh(),
        out_type=[
            jax.ShapeDtypeStruct((PADN, HW), jnp.float32),
            jax.ShapeDtypeStruct((PADN,), jnp.float32),
        ],
        scratch_types=[
            pltpu.VMEM((128,), jnp.int32),
            pltpu.VMEM((128,), jnp.float32),
            pltpu.VMEM((128, HW), jnp.float32),
            pltpu.VMEM((128,), jnp.float32),
            pltpu.SemaphoreType.DMA,
        ],
    )
    return f(pos_cm, ws_cm, xbits)


def _mlp_bf16(xb, gw, uw, dw):
    g = lax.dot_general(xb, gw, (((1,), (1,)), ((), ())),
                        preferred_element_type=jnp.float32)
    u = lax.dot_general(xb, uw, (((1,), (1,)), ((), ())),
                        preferred_element_type=jnp.float32)
    act = 0.5 * g * (1.0 + lax.erf(g * 0.7071067811865476))
    h = (act * u).astype(jnp.bfloat16)
    return lax.dot_general(h, dw, (((1,), (1,)), ((), ())),
                           preferred_element_type=jnp.float32)


def _experts_body(offs_ref, xs_ref, w_ref, eg_ref, eu_ref, ed_ref, ys_ref,
                  xbuf, ybuf, in_sem, out_sem):
    s = pl.program_id(0)
    startr = pl.multiple_of(offs_ref[s], 16)
    endr = offs_ref[s + 1]
    nblk = (endr - startr + BLK - 1) // BLK

    def row0(b):
        return pl.multiple_of(startr + b * BLK, 16)

    def in_copy(b, slot):
        return pltpu.make_async_copy(
            xs_ref.at[pl.ds(row0(b), BLK)], xbuf.at[slot], in_sem.at[slot])

    def out_copy(b, slot):
        return pltpu.make_async_copy(
            ybuf.at[slot], ys_ref.at[pl.ds(row0(b), BLK)], out_sem.at[slot])

    @pl.when(nblk > 0)
    def _():
        in_copy(0, 0).start()

    def body(b, _):
        slot = lax.rem(b, 2)
        in_copy(b, slot).wait()

        @pl.when(b + 1 < nblk)
        def _():
            in_copy(b + 1, 1 - slot).start()

        y = _mlp_bf16(xbuf[slot], eg_ref[0], eu_ref[0], ed_ref[0])
        y = y * w_ref[pl.ds(row0(b), BLK), :]

        @pl.when(b >= 2)
        def _():
            out_copy(b - 2, slot).wait()

        ybuf[slot] = y
        out_copy(b, slot).start()
        return 0

    lax.fori_loop(0, nblk, body, 0)

    @pl.when(nblk >= 1)
    def _():
        out_copy(nblk - 1, lax.rem(nblk - 1, 2)).wait()

    @pl.when(nblk >= 2)
    def _():
        out_copy(nblk - 2, lax.rem(nblk - 2, 2)).wait()


def _experts(offs_ext, xs_bf, wsrt2, eg_all, eu_all, ed_all):
    return pl.pallas_call(
        _experts_body,
        grid=(E + 1,),
        in_specs=[
            pl.BlockSpec(memory_space=pltpu.SMEM),
            pl.BlockSpec(memory_space=pl.ANY),
            pl.BlockSpec((PADN, 1), lambda s: (0, 0)),
            pl.BlockSpec((1, I, H), lambda s: (s, 0, 0)),
            pl.BlockSpec((1, I, H), lambda s: (s, 0, 0)),
            pl.BlockSpec((1, H, I), lambda s: (s, 0, 0)),
        ],
        out_specs=pl.BlockSpec(memory_space=pl.ANY),
        out_shape=jax.ShapeDtypeStruct((PADN, H), jnp.float32),
        scratch_shapes=[
            pltpu.VMEM((2, BLK, H), jnp.bfloat16),
            pltpu.VMEM((2, BLK, H), jnp.float32),
            pltpu.SemaphoreType.DMA((2,)),
            pltpu.SemaphoreType.DMA((2,)),
        ],
        compiler_params=pltpu.CompilerParams(
            dimension_semantics=("arbitrary",),
        ),
    )(offs_ext, xs_bf, wsrt2, eg_all, eu_all, ed_all)


_CCH = 8


def _combine_body(ys, inv, out, iv16, sbuf, gbuf, sem):
    cid = lax.axis_index("c")
    sid = lax.axis_index("s")
    base = cid * (N // 2) + sid * (N // 32)

    def chunk(cc, _):
        tb = base + cc * _CCH
        pltpu.sync_copy(inv.at[pl.ds(tb, _CCH)], iv16.at[pl.ds(0, _CCH)])
        pltpu.sync_copy(inv.at[pl.ds(N + tb, _CCH)], iv16.at[pl.ds(_CCH, _CCH)])
        pltpu.sync_copy(ys.at[pl.ds(P0 + tb, _CCH)], sbuf)
        pltpu.async_copy(ys.at[iv16], gbuf, sem).wait()
        for r in range(_CCH):
            for c in range(H // 16):
                sl = pl.ds(c * 16, 16)
                sbuf[r, sl] = sbuf[r, sl] + gbuf[r, sl] + gbuf[r + _CCH, sl]
        pltpu.sync_copy(sbuf, out.at[pl.ds(tb, _CCH)])
        return 0

    lax.fori_loop(0, 8, chunk, 0)


def _combine(ys, inv):
    f = pl.kernel(
        _combine_body,
        mesh=_sc_mesh(),
        out_type=jax.ShapeDtypeStruct((N, H), jnp.float32),
        scratch_types=[
            pltpu.VMEM((2 * _CCH,), jnp.int32),
            pltpu.VMEM((_CCH, H), jnp.float32),
            pltpu.VMEM((2 * _CCH, H), jnp.float32),
            pltpu.SemaphoreType.DMA,
        ],
    )
    return f(ys, inv)


def _moe(x, gwp, eg_all, eu_all, ed_all):
    pos2, w2, offs8 = _gate(x, gwp)
    pos_cm = pos2.T.reshape(-1)
    ws_cm = w2.T.reshape(-1)
    xb = x.astype(jnp.bfloat16)
    xbits = lax.bitcast_convert_type(xb.reshape(N, HW, 2), jnp.float32)
    xs_bits, wsrt = _distribute(pos_cm, ws_cm, xbits)
    xs_bf = lax.bitcast_convert_type(xs_bits, jnp.bfloat16).reshape(PADN, H)
    offs_ext = jnp.concatenate(
        [offs8[0], jnp.array([P0, NTOT], jnp.int32)]).astype(jnp.int32)
    wsrt2 = wsrt.reshape(PADN, 1)
    ys = _experts(offs_ext, xs_bf, wsrt2, eg_all, eu_all, ed_all)
    return _combine(ys, pos_cm)


def kernel(hidden_states, gate_weight, expert_gate_w, expert_up_w,
           expert_down_w, shared_gate_w, shared_up_w, shared_down_w):
    b, s, h = hidden_states.shape
    x = hidden_states.reshape(-1, h).astype(jnp.float32)
    gwp = jnp.zeros((128, h), jnp.float32).at[:E].set(gate_weight)
    eg_all = jnp.concatenate(
        [expert_gate_w, shared_gate_w[None]], axis=0).astype(jnp.bfloat16)
    eu_all = jnp.concatenate(
        [expert_up_w, shared_up_w[None]], axis=0).astype(jnp.bfloat16)
    ed_all = jnp.concatenate(
        [expert_down_w, shared_down_w[None]], axis=0).astype(jnp.bfloat16)
    out = _moe(x, gwp, eg_all, eu_all, ed_all)
    return out.reshape(b, s, h)


# speedup vs baseline: 1.6669x; 1.6669x over previous
import jax
import jax.numpy as jnp
from jax import lax
from jax.experimental import pallas as pl
from jax.experimental.pallas import tpu as pltpu
from jax.experimental.pallas import tpu_sc as plsc

N, H, E, K, I = 2048, 1024, 16, 2, 512
NSLOT = N * K
P0 = NSLOT + 256
NTOT = P0 + N
BLK = 256
PADN = NTOT + BLK
HW = H // 2
NEG = -1e30
INTMAX = 2147483647


def _gate_body(x_ref, gw_ref, pos_ref, w_ref, offs_ref):
    logits = lax.dot_general(x_ref[...], gw_ref[...], (((1,), (1,)), ((), ())),
                             preferred_element_type=jnp.float32)
    iota = lax.broadcasted_iota(jnp.int32, (N, 128), 1)
    l = jnp.where(iota < E, logits, NEG)
    m = jnp.max(l, axis=1, keepdims=True)
    z = jnp.sum(jnp.exp(l - m), axis=1, keepdims=True)
    a1 = jnp.min(jnp.where(l == m, iota, INTMAX), axis=1, keepdims=True)
    l2 = jnp.where(iota == a1, NEG, l)
    m2 = jnp.max(l2, axis=1, keepdims=True)
    a2 = jnp.min(jnp.where(l2 == m2, iota, INTMAX), axis=1, keepdims=True)
    s1 = 1.0 / z
    s2 = jnp.exp(m2 - m) / z
    denom = s1 + s2 + 1e-20
    w_ref[...] = jnp.concatenate([s1 / denom, s2 / denom], axis=1)

    ecol = lax.broadcasted_iota(jnp.int32, (N, E), 1)
    oh1 = (a1 == ecol).astype(jnp.float32)
    oh2 = (a2 == ecol).astype(jnp.float32)
    tri_r = lax.broadcasted_iota(jnp.int32, (N, N), 0)
    tri_c = lax.broadcasted_iota(jnp.int32, (N, N), 1)
    stl = (tri_c < tri_r).astype(jnp.float32)
    pre1 = lax.dot_general(stl, oh1, (((1,), (0,)), ((), ())),
                           preferred_element_type=jnp.float32)
    pre2 = lax.dot_general(stl, oh2, (((1,), (0,)), ((), ())),
                           preferred_element_type=jnp.float32)
    tot1 = jnp.sum(oh1, axis=0, keepdims=True)
    tot2 = jnp.sum(oh2, axis=0, keepdims=True)
    counts = tot1 + tot2
    c16 = jnp.float32(16.0) * jnp.ceil(counts / 16.0)
    e_r = lax.broadcasted_iota(jnp.int32, (E, E), 0)
    e_c = lax.broadcasted_iota(jnp.int32, (E, E), 1)
    stl16 = (e_r < e_c).astype(jnp.float32)
    excl = lax.dot_general(c16, stl16, (((1,), (0,)), ((), ())),
                           preferred_element_type=jnp.float32)
    rank1 = jnp.sum(oh1 * pre1, axis=1, keepdims=True)
    rank2 = jnp.sum(oh2 * pre2, axis=1, keepdims=True)
    base1 = jnp.sum(oh1 * excl, axis=1, keepdims=True)
    base2 = jnp.sum(oh2 * (excl + tot1), axis=1, keepdims=True)
    pos1 = (base1 + rank1).astype(jnp.int32)
    pos2 = (base2 + rank2).astype(jnp.int32)
    pos_ref[...] = jnp.concatenate([pos1, pos2], axis=1)
    offs_ref[...] = jnp.broadcast_to(excl, (8, E)).astype(jnp.int32)


def _gate(x, gwp):
    return pl.pallas_call(
        _gate_body,
        in_specs=[pl.BlockSpec((N, H), lambda: (0, 0)),
                  pl.BlockSpec((128, H), lambda: (0, 0))],
        out_specs=[pl.BlockSpec((N, 2), lambda: (0, 0)),
                   pl.BlockSpec((N, 2), lambda: (0, 0)),
                   pl.BlockSpec((8, E), lambda: (0, 0))],
        out_shape=[jax.ShapeDtypeStruct((N, 2), jnp.int32),
                   jax.ShapeDtypeStruct((N, 2), jnp.float32),
                   jax.ShapeDtypeStruct((8, E), jnp.int32)],
    )(x, gwp)


_MESH_CACHE = []


def _sc_mesh():
    if not _MESH_CACHE:
        _MESH_CACHE.append(
            plsc.VectorSubcoreMesh(core_axis_name="c", subcore_axis_name="s"))
    return _MESH_CACHE[0]


def _dist_body(pos, ws, x, xs, wsrt, pos_loc, ws_loc, xloc, ones_loc, sem):
    cid = lax.axis_index("c")
    sid = lax.axis_index("s")
    w = cid * 16 + sid
    sbase = w * 128
    tok0 = lax.rem(w, 16) * 128

    pltpu.sync_copy(ws.at[pl.ds(sbase, 128)], ws_loc)
    for j in range(8):
        ones_loc[pl.ds(j * 16, 16)] = jnp.full((16,), 1.0, jnp.float32)

    for h in range(2):
        th = tok0 + h * 64
        pltpu.sync_copy(pos.at[pl.ds(sbase + h * 64, 64)], pos_loc)
        pltpu.sync_copy(x.at[pl.ds(th, 64)], xloc)
        cp_x = pltpu.make_async_copy(xloc, xs.at[pos_loc], sem)
        cp_x.start()
        cp_w = pltpu.make_async_copy(ws_loc.at[pl.ds(h * 64, 64)],
                                     wsrt.at[pos_loc], sem)
        cp_w.start()

        @pl.when(w >= 16)
        def _():
            pltpu.sync_copy(xloc, xs.at[pl.ds(P0 + th, 64)])
            pltpu.sync_copy(ones_loc.at[pl.ds(0, 64)],
                            wsrt.at[pl.ds(P0 + th, 64)])

        cp_x.wait()
        cp_w.wait()


def _distribute(pos_cm, ws_cm, x):
    f = pl.kernel(
        _dist_body,
        mesh=_sc_mesh(),
        out_type=[
            jax.ShapeDtypeStruct((PADN, H), jnp.float32),
            jax.ShapeDtypeStruct((PADN,), jnp.float32),
        ],
        scratch_types=[
            pltpu.VMEM((64,), jnp.int32),
            pltpu.VMEM((128,), jnp.float32),
            pltpu.VMEM((64, H), jnp.float32),
            pltpu.VMEM((128,), jnp.float32),
            pltpu.SemaphoreType.DMA,
        ],
    )
    return f(pos_cm, ws_cm, x)


def _mlp_bf16(xb, gw, uw, dw):
    g = lax.dot_general(xb, gw, (((1,), (1,)), ((), ())),
                        preferred_element_type=jnp.float32)
    u = lax.dot_general(xb, uw, (((1,), (1,)), ((), ())),
                        preferred_element_type=jnp.float32)
    act = 0.5 * g * (1.0 + lax.erf(g * 0.7071067811865476))
    h = (act * u).astype(jnp.bfloat16)
    return lax.dot_general(h, dw, (((1,), (1,)), ((), ())),
                           preferred_element_type=jnp.float32)


def _experts_body(offs_ref, xs_ref, w_ref, eg_ref, eu_ref, ed_ref, ys_ref,
                  xbuf, ybuf, in_sem, out_sem):
    s = pl.program_id(0)
    startr = pl.multiple_of(offs_ref[s], 16)
    endr = offs_ref[s + 1]
    nblk = (endr - startr + BLK - 1) // BLK

    def row0(b):
        return pl.multiple_of(startr + b * BLK, 16)

    def in_copy(b, slot):
        return pltpu.make_async_copy(
            xs_ref.at[pl.ds(row0(b), BLK)], xbuf.at[slot], in_sem.at[slot])

    def out_copy(b, slot):
        return pltpu.make_async_copy(
            ybuf.at[slot], ys_ref.at[pl.ds(row0(b), BLK)], out_sem.at[slot])

    @pl.when(nblk > 0)
    def _():
        in_copy(0, 0).start()

    def body(b, _):
        slot = lax.rem(b, 2)
        in_copy(b, slot).wait()

        @pl.when(b + 1 < nblk)
        def _():
            in_copy(b + 1, 1 - slot).start()

        y = _mlp_bf16(xbuf[slot].astype(jnp.bfloat16),
                      eg_ref[0], eu_ref[0], ed_ref[0])
        y = y * w_ref[pl.ds(row0(b), BLK), :]

        @pl.when(b >= 2)
        def _():
            out_copy(b - 2, slot).wait()

        ybuf[slot] = y
        out_copy(b, slot).start()
        return 0

    lax.fori_loop(0, nblk, body, 0)

    @pl.when(nblk >= 1)
    def _():
        out_copy(nblk - 1, lax.rem(nblk - 1, 2)).wait()

    @pl.when(nblk >= 2)
    def _():
        out_copy(nblk - 2, lax.rem(nblk - 2, 2)).wait()


def _experts(offs_ext, xs_bf, wsrt2, eg_all, eu_all, ed_all):
    return pl.pallas_call(
        _experts_body,
        grid=(E + 1,),
        in_specs=[
            pl.BlockSpec(memory_space=pltpu.SMEM),
            pl.BlockSpec(memory_space=pl.ANY),
            pl.BlockSpec((PADN, 1), lambda s: (0, 0)),
            pl.BlockSpec((1, I, H), lambda s: (s, 0, 0)),
            pl.BlockSpec((1, I, H), lambda s: (s, 0, 0)),
            pl.BlockSpec((1, H, I), lambda s: (s, 0, 0)),
        ],
        out_specs=pl.BlockSpec(memory_space=pl.ANY),
        out_shape=jax.ShapeDtypeStruct((PADN, H), jnp.float32),
        scratch_shapes=[
            pltpu.VMEM((2, BLK, H), jnp.float32),
            pltpu.VMEM((2, BLK, H), jnp.float32),
            pltpu.SemaphoreType.DMA((2,)),
            pltpu.SemaphoreType.DMA((2,)),
        ],
        compiler_params=pltpu.CompilerParams(
            dimension_semantics=("arbitrary",),
        ),
    )(offs_ext, xs_bf, wsrt2, eg_all, eu_all, ed_all)


_CCH = 8


def _combine_body(ys, inv, out, iv16, sbuf, gbuf, sem):
    cid = lax.axis_index("c")
    sid = lax.axis_index("s")
    base = cid * (N // 2) + sid * (N // 32)

    def chunk(cc, _):
        tb = base + cc * _CCH
        pltpu.sync_copy(inv.at[pl.ds(tb, _CCH)], iv16.at[pl.ds(0, _CCH)])
        pltpu.sync_copy(inv.at[pl.ds(N + tb, _CCH)], iv16.at[pl.ds(_CCH, _CCH)])
        pltpu.sync_copy(ys.at[pl.ds(P0 + tb, _CCH)], sbuf)
        pltpu.async_copy(ys.at[iv16], gbuf, sem).wait()
        for r in range(_CCH):
            for c in range(H // 16):
                sl = pl.ds(c * 16, 16)
                sbuf[r, sl] = sbuf[r, sl] + gbuf[r, sl] + gbuf[r + _CCH, sl]
        pltpu.sync_copy(sbuf, out.at[pl.ds(tb, _CCH)])
        return 0

    lax.fori_loop(0, 8, chunk, 0)


def _combine(ys, inv):
    f = pl.kernel(
        _combine_body,
        mesh=_sc_mesh(),
        out_type=jax.ShapeDtypeStruct((N, H), jnp.float32),
        scratch_types=[
            pltpu.VMEM((2 * _CCH,), jnp.int32),
            pltpu.VMEM((_CCH, H), jnp.float32),
            pltpu.VMEM((2 * _CCH, H), jnp.float32),
            pltpu.SemaphoreType.DMA,
        ],
    )
    return f(ys, inv)


def _moe(x, gwp, eg_all, eu_all, ed_all):
    pos2, w2, offs8 = _gate(x, gwp)
    pos_cm = pos2.T.reshape(-1)
    ws_cm = w2.T.reshape(-1)
    xs, wsrt = _distribute(pos_cm, ws_cm, x)
    offs_ext = jnp.concatenate(
        [offs8[0], jnp.array([P0, NTOT], jnp.int32)]).astype(jnp.int32)
    wsrt2 = wsrt.reshape(PADN, 1)
    ys = _experts(offs_ext, xs, wsrt2, eg_all, eu_all, ed_all)
    return _combine(ys, pos_cm)


def kernel(hidden_states, gate_weight, expert_gate_w, expert_up_w,
           expert_down_w, shared_gate_w, shared_up_w, shared_down_w):
    b, s, h = hidden_states.shape
    x = hidden_states.reshape(-1, h).astype(jnp.float32)
    gwp = jnp.zeros((128, h), jnp.float32).at[:E].set(gate_weight)
    eg_all = jnp.concatenate(
        [expert_gate_w, shared_gate_w[None]], axis=0).astype(jnp.bfloat16)
    eu_all = jnp.concatenate(
        [expert_up_w, shared_up_w[None]], axis=0).astype(jnp.bfloat16)
    ed_all = jnp.concatenate(
        [expert_down_w, shared_down_w[None]], axis=0).astype(jnp.bfloat16)
    out = _moe(x, gwp, eg_all, eu_all, ed_all)
    return out.reshape(b, s, h)


# speedup vs baseline: 1.9061x; 1.1435x over previous
import jax
import jax.numpy as jnp
from jax import lax
from jax.experimental import pallas as pl
from jax.experimental.pallas import tpu as pltpu
from jax.experimental.pallas import tpu_sc as plsc

N, H, E, K, I = 2048, 1024, 16, 2, 512
NSLOT = N * K
P0 = NSLOT + 256
NTOT = P0 + N
BLK = 256
PADN = NTOT + BLK
HW = H // 2
NEG = -1e30
INTMAX = 2147483647


def _gate_body(x_ref, gw_ref, pos_ref, w_ref, offs_ref):
    logits = lax.dot_general(x_ref[...], gw_ref[...], (((1,), (1,)), ((), ())),
                             preferred_element_type=jnp.float32)
    iota = lax.broadcasted_iota(jnp.int32, (N, 128), 1)
    l = jnp.where(iota < E, logits, NEG)
    m = jnp.max(l, axis=1, keepdims=True)
    z = jnp.sum(jnp.exp(l - m), axis=1, keepdims=True)
    a1 = jnp.min(jnp.where(l == m, iota, INTMAX), axis=1, keepdims=True)
    l2 = jnp.where(iota == a1, NEG, l)
    m2 = jnp.max(l2, axis=1, keepdims=True)
    a2 = jnp.min(jnp.where(l2 == m2, iota, INTMAX), axis=1, keepdims=True)
    s1 = 1.0 / z
    s2 = jnp.exp(m2 - m) / z
    denom = s1 + s2 + 1e-20
    w_ref[...] = jnp.concatenate([s1 / denom, s2 / denom], axis=1)

    ecol = lax.broadcasted_iota(jnp.int32, (N, E), 1)
    oh1 = (a1 == ecol).astype(jnp.float32)
    oh2 = (a2 == ecol).astype(jnp.float32)
    tri_r = lax.broadcasted_iota(jnp.int32, (N, N), 0)
    tri_c = lax.broadcasted_iota(jnp.int32, (N, N), 1)
    stl = (tri_c < tri_r).astype(jnp.float32)
    pre1 = lax.dot_general(stl, oh1, (((1,), (0,)), ((), ())),
                           preferred_element_type=jnp.float32)
    pre2 = lax.dot_general(stl, oh2, (((1,), (0,)), ((), ())),
                           preferred_element_type=jnp.float32)
    tot1 = jnp.sum(oh1, axis=0, keepdims=True)
    tot2 = jnp.sum(oh2, axis=0, keepdims=True)
    counts = tot1 + tot2
    c16 = jnp.float32(16.0) * jnp.ceil(counts / 16.0)
    e_r = lax.broadcasted_iota(jnp.int32, (E, E), 0)
    e_c = lax.broadcasted_iota(jnp.int32, (E, E), 1)
    stl16 = (e_r < e_c).astype(jnp.float32)
    excl = lax.dot_general(c16, stl16, (((1,), (0,)), ((), ())),
                           preferred_element_type=jnp.float32)
    rank1 = jnp.sum(oh1 * pre1, axis=1, keepdims=True)
    rank2 = jnp.sum(oh2 * pre2, axis=1, keepdims=True)
    base1 = jnp.sum(oh1 * excl, axis=1, keepdims=True)
    base2 = jnp.sum(oh2 * (excl + tot1), axis=1, keepdims=True)
    pos1 = (base1 + rank1).astype(jnp.int32)
    pos2 = (base2 + rank2).astype(jnp.int32)
    pos_ref[...] = jnp.concatenate([pos1, pos2], axis=1)
    offs_ref[...] = jnp.broadcast_to(excl, (8, E)).astype(jnp.int32)


def _gate(x, gwp):
    return pl.pallas_call(
        _gate_body,
        in_specs=[pl.BlockSpec((N, H), lambda: (0, 0)),
                  pl.BlockSpec((128, H), lambda: (0, 0))],
        out_specs=[pl.BlockSpec((N, 2), lambda: (0, 0)),
                   pl.BlockSpec((N, 2), lambda: (0, 0)),
                   pl.BlockSpec((8, E), lambda: (0, 0))],
        out_shape=[jax.ShapeDtypeStruct((N, 2), jnp.int32),
                   jax.ShapeDtypeStruct((N, 2), jnp.float32),
                   jax.ShapeDtypeStruct((8, E), jnp.int32)],
    )(x, gwp)


_MESH_CACHE = []


def _sc_mesh():
    if not _MESH_CACHE:
        _MESH_CACHE.append(
            plsc.VectorSubcoreMesh(core_axis_name="c", subcore_axis_name="s"))
    return _MESH_CACHE[0]


def _dist_body(pos, ws, x, xs, wsrt, pos_loc, ws_loc, xloc, ones_loc, sem):
    cid = lax.axis_index("c")
    sid = lax.axis_index("s")
    w = cid * 16 + sid
    sbase = w * 128
    tok0 = lax.rem(w, 16) * 128

    pltpu.sync_copy(ws.at[pl.ds(sbase, 128)], ws_loc)
    for j in range(8):
        ones_loc[pl.ds(j * 16, 16)] = jnp.full((16,), 1.0, jnp.float32)

    cps = [None, None]
    for q in range(4):
        b = q % 2
        th = tok0 + q * 32
        if cps[b] is not None:
            for cp in cps[b]:
                cp.wait()
        pltpu.sync_copy(pos.at[pl.ds(sbase + q * 32, 32)], pos_loc.at[b])
        pltpu.sync_copy(x.at[pl.ds(th, 32)], xloc.at[b])
        cp_x = pltpu.make_async_copy(xloc.at[b], xs.at[pos_loc.at[b]],
                                     sem.at[b])
        cp_x.start()
        cp_w = pltpu.make_async_copy(ws_loc.at[pl.ds(q * 32, 32)],
                                     wsrt.at[pos_loc.at[b]], sem.at[b])
        cp_w.start()

        @pl.when(w >= 16)
        def _():
            pltpu.sync_copy(xloc.at[b], xs.at[pl.ds(P0 + th, 32)])
            pltpu.sync_copy(ones_loc.at[pl.ds(0, 32)],
                            wsrt.at[pl.ds(P0 + th, 32)])

        cps[b] = [cp_x, cp_w]
    for cp in cps[0] + cps[1]:
        cp.wait()


def _distribute(pos_cm, ws_cm, x):
    f = pl.kernel(
        _dist_body,
        mesh=_sc_mesh(),
        out_type=[
            jax.ShapeDtypeStruct((PADN, H), jnp.float32),
            jax.ShapeDtypeStruct((PADN,), jnp.float32),
        ],
        scratch_types=[
            pltpu.VMEM((2, 32), jnp.int32),
            pltpu.VMEM((128,), jnp.float32),
            pltpu.VMEM((2, 32, H), jnp.float32),
            pltpu.VMEM((128,), jnp.float32),
            pltpu.SemaphoreType.DMA((2,)),
        ],
    )
    return f(pos_cm, ws_cm, x)


def _mlp_bf16(xb, gw, uw, dw):
    g = lax.dot_general(xb, gw, (((1,), (1,)), ((), ())),
                        preferred_element_type=jnp.float32)
    u = lax.dot_general(xb, uw, (((1,), (1,)), ((), ())),
                        preferred_element_type=jnp.float32)
    act = 0.5 * g * (1.0 + lax.erf(g * 0.7071067811865476))
    h = (act * u).astype(jnp.bfloat16)
    return lax.dot_general(h, dw, (((1,), (1,)), ((), ())),
                           preferred_element_type=jnp.float32)


def _experts_body(offs_ref, xs_ref, w_ref, eg_ref, eu_ref, ed_ref, ys_ref,
                  xbuf, ybuf, in_sem, out_sem):
    s = pl.program_id(0)
    startr = pl.multiple_of(offs_ref[s], 16)
    endr = offs_ref[s + 1]
    nblk = (endr - startr + BLK - 1) // BLK

    def row0(b):
        return pl.multiple_of(startr + b * BLK, 16)

    def in_copy(b, slot):
        return pltpu.make_async_copy(
            xs_ref.at[pl.ds(row0(b), BLK)], xbuf.at[slot], in_sem.at[slot])

    def out_copy(b, slot):
        return pltpu.make_async_copy(
            ybuf.at[slot], ys_ref.at[pl.ds(row0(b), BLK)], out_sem.at[slot])

    @pl.when(nblk > 0)
    def _():
        in_copy(0, 0).start()

    def body(b, _):
        slot = lax.rem(b, 2)
        in_copy(b, slot).wait()

        @pl.when(b + 1 < nblk)
        def _():
            in_copy(b + 1, 1 - slot).start()

        y = _mlp_bf16(xbuf[slot].astype(jnp.bfloat16),
                      eg_ref[0], eu_ref[0], ed_ref[0])
        y = y * w_ref[pl.ds(row0(b), BLK), :]

        @pl.when(b >= 2)
        def _():
            out_copy(b - 2, slot).wait()

        ybuf[slot] = y
        out_copy(b, slot).start()
        return 0

    lax.fori_loop(0, nblk, body, 0)

    @pl.when(nblk >= 1)
    def _():
        out_copy(nblk - 1, lax.rem(nblk - 1, 2)).wait()

    @pl.when(nblk >= 2)
    def _():
        out_copy(nblk - 2, lax.rem(nblk - 2, 2)).wait()


def _experts(offs_ext, xs_bf, wsrt2, eg_all, eu_all, ed_all):
    return pl.pallas_call(
        _experts_body,
        grid=(E + 1,),
        in_specs=[
            pl.BlockSpec(memory_space=pltpu.SMEM),
            pl.BlockSpec(memory_space=pl.ANY),
            pl.BlockSpec((PADN, 1), lambda s: (0, 0)),
            pl.BlockSpec((1, I, H), lambda s: (s, 0, 0)),
            pl.BlockSpec((1, I, H), lambda s: (s, 0, 0)),
            pl.BlockSpec((1, H, I), lambda s: (s, 0, 0)),
        ],
        out_specs=pl.BlockSpec(memory_space=pl.ANY),
        out_shape=jax.ShapeDtypeStruct((PADN, H), jnp.float32),
        scratch_shapes=[
            pltpu.VMEM((2, BLK, H), jnp.float32),
            pltpu.VMEM((2, BLK, H), jnp.float32),
            pltpu.SemaphoreType.DMA((2,)),
            pltpu.SemaphoreType.DMA((2,)),
        ],
        compiler_params=pltpu.CompilerParams(
            dimension_semantics=("arbitrary",),
        ),
    )(offs_ext, xs_bf, wsrt2, eg_all, eu_all, ed_all)


_CCH = 16


def _combine_body(ys, inv, out, iv_e, iv_o, sbuf, ge, go, sem, sem2, sem3):
    cid = lax.axis_index("c")
    sid = lax.axis_index("s")
    base = cid * (N // 2) + sid * (N // 32)
    pltpu.sync_copy(inv.at[pl.ds(base, 64)], iv_e)
    pltpu.sync_copy(inv.at[pl.ds(N + base, 64)], iv_o)

    def chunk(cc, _):
        tb = base + cc * _CCH
        ive = iv_e[pl.ds(cc * _CCH, _CCH)]
        ivo = iv_o[pl.ds(cc * _CCH, _CCH)]
        cs = pltpu.make_async_copy(ys.at[pl.ds(P0 + tb, _CCH)], sbuf, sem)
        ce = pltpu.make_async_copy(ys.at[ive], ge, sem2)
        co = pltpu.make_async_copy(ys.at[ivo], go, sem3)
        cs.start()
        ce.start()
        co.start()
        cs.wait()
        ce.wait()
        co.wait()

        def addrow(r, _):
            for c in range(H // 16):
                sl = pl.ds(c * 16, 16)
                sbuf[r, sl] = sbuf[r, sl] + ge[r, sl] + go[r, sl]
            return 0

        lax.fori_loop(0, _CCH, addrow, 0)
        pltpu.sync_copy(sbuf, out.at[pl.ds(tb, _CCH)])
        return 0

    lax.fori_loop(0, 4, chunk, 0)


def _combine(ys, inv):
    f = pl.kernel(
        _combine_body,
        mesh=_sc_mesh(),
        out_type=jax.ShapeDtypeStruct((N, H), jnp.float32),
        scratch_types=[
            pltpu.VMEM((64,), jnp.int32),
            pltpu.VMEM((64,), jnp.int32),
            pltpu.VMEM((_CCH, H), jnp.float32),
            pltpu.VMEM((_CCH, H), jnp.float32),
            pltpu.VMEM((_CCH, H), jnp.float32),
            pltpu.SemaphoreType.DMA,
            pltpu.SemaphoreType.DMA,
            pltpu.SemaphoreType.DMA,
        ],
    )
    return f(ys, inv)


def _moe(x, gwp, eg_all, eu_all, ed_all):
    pos2, w2, offs8 = _gate(x, gwp)
    pos_cm = pos2.T.reshape(-1)
    ws_cm = w2.T.reshape(-1)
    xs, wsrt = _distribute(pos_cm, ws_cm, x)
    offs_ext = jnp.concatenate(
        [offs8[0], jnp.array([P0, NTOT], jnp.int32)]).astype(jnp.int32)
    wsrt2 = wsrt.reshape(PADN, 1)
    ys = _experts(offs_ext, xs, wsrt2, eg_all, eu_all, ed_all)
    return _combine(ys, pos_cm)


def kernel(hidden_states, gate_weight, expert_gate_w, expert_up_w,
           expert_down_w, shared_gate_w, shared_up_w, shared_down_w):
    b, s, h = hidden_states.shape
    x = hidden_states.reshape(-1, h).astype(jnp.float32)
    gwp = jnp.zeros((128, h), jnp.float32).at[:E].set(gate_weight)
    eg_all = jnp.concatenate(
        [expert_gate_w, shared_gate_w[None]], axis=0).astype(jnp.bfloat16)
    eu_all = jnp.concatenate(
        [expert_up_w, shared_up_w[None]], axis=0).astype(jnp.bfloat16)
    ed_all = jnp.concatenate(
        [expert_down_w, shared_down_w[None]], axis=0).astype(jnp.bfloat16)
    out = _moe(x, gwp, eg_all, eu_all, ed_all)
    return out.reshape(b, s, h)
